# baseline (speedup 1.0000x reference)
import functools

import jax
import jax.numpy as jnp
from jax import lax
from jax.experimental import pallas as pl
from jax.experimental.pallas import tpu as pltpu
from jax.experimental.pallas import tpu_sc as plsc

N = 10000
E = 320000
D = 128
BN_EPS = 1e-5

NC = 2
NS = 16
NW = NC * NS
CHUNK = 128
NCK = 80
EPAD = NW * NCK * CHUNK
NR = N + CHUNK
RPS = N // NS
DEG_W = 16
NPAD = 10240
DROWS = NPAD // NS
NSLOT = 5

_mesh = plsc.VectorSubcoreMesh(core_axis_name="c", subcore_axis_name="s")
_sc_params = pltpu.CompilerParams(use_tc_tiling_on_sc=False)


@functools.partial(
    pl.kernel,
    out_type=jax.ShapeDtypeStruct((NC, NPAD, DEG_W), jnp.float32),
    mesh=_mesh,
    scratch_types=[
        pltpu.VMEM((NCK, CHUNK), jnp.int32),
        pltpu.VMEM((CHUNK, DEG_W), jnp.float32),
        pltpu.VMEM_SHARED((NPAD, DEG_W), jnp.float32),
        pltpu.SemaphoreType.DMA,
    ],
    compiler_params=_sc_params,
)
def _deg_kernel(dstR_hbm, ones_hbm, zeros_hbm, out_hbm, dst_v, ones_v, acc,
                sem):
    c = lax.axis_index("c")
    s = lax.axis_index("s")
    wid = c * NS + s

    pltpu.sync_copy(ones_hbm, ones_v)
    pltpu.sync_copy(dstR_hbm.at[wid], dst_v)
    r0 = s * DROWS
    pltpu.sync_copy(zeros_hbm.at[pl.ds(r0, DROWS)], acc.at[pl.ds(r0, DROWS)])
    plsc.subcore_barrier()

    @pl.loop(0, NCK // 8)
    def _wave(w):
        i0 = w * 8
        for k in range(8):
            pltpu.async_copy(ones_v, acc.at[dst_v.at[i0 + k]], sem, add=True)
        for k in range(8):
            pltpu.make_async_copy(ones_v, acc.at[dst_v.at[i0 + k]], sem).wait()

    plsc.subcore_barrier()
    pltpu.sync_copy(acc.at[pl.ds(r0, DROWS)], out_hbm.at[c, pl.ds(r0, DROWS)])


@functools.partial(
    pl.kernel,
    out_type=jax.ShapeDtypeStruct((NC, N, D), jnp.bfloat16),
    mesh=_mesh,
    scratch_types=[
        pltpu.VMEM((NCK, CHUNK), jnp.int32),
        pltpu.VMEM((NCK, CHUNK), jnp.int32),
        [pltpu.VMEM((CHUNK, D), jnp.bfloat16) for _ in range(NSLOT)],
        pltpu.VMEM_SHARED((NR, D), jnp.bfloat16),
        [pltpu.SemaphoreType.DMA for _ in range(NSLOT)],
        [pltpu.SemaphoreType.DMA for _ in range(NSLOT)],
    ],
    compiler_params=_sc_params,
)
def _agg_kernel(srcR_hbm, dstR_hbm, hp_hbm, out_hbm,
                src_v, dst_v, rows, acc, gsem, ssem):
    c = lax.axis_index("c")
    s = lax.axis_index("s")
    wid = c * NS + s

    zero32 = jnp.zeros((32,), jnp.bfloat16)

    @pl.loop(0, CHUNK)
    def _zfill(r):
        for k in range(D // 32):
            rows[0][r, pl.ds(k * 32, 32)] = zero32

    r0 = s * RPS
    for k in range(4):
        pltpu.sync_copy(rows[0], acc.at[pl.ds(r0 + k * CHUNK, CHUNK)])
    pltpu.sync_copy(rows[0].at[pl.ds(0, RPS - 4 * CHUNK)],
                    acc.at[pl.ds(r0 + 4 * CHUNK, RPS - 4 * CHUNK)])

    pltpu.sync_copy(srcR_hbm.at[wid], src_v)
    pltpu.sync_copy(dstR_hbm.at[wid], dst_v)

    for b in range(NSLOT - 1):
        pltpu.async_copy(hp_hbm.at[src_v.at[b]], rows[b], gsem[b])
    plsc.subcore_barrier()

    def _prepare(i_next, b_next, wait_prev):
        if wait_prev:
            pltpu.make_async_copy(rows[b_next],
                                  acc.at[dst_v.at[i_next]],
                                  ssem[b_next]).wait()
        pltpu.async_copy(hp_hbm.at[src_v.at[i_next]], rows[b_next],
                         gsem[b_next])

    @pl.loop(0, NCK // NSLOT)
    def _group(j):
        base = j * NSLOT
        for b in range(NSLOT):
            i = base + b
            pltpu.make_async_copy(hp_hbm.at[src_v.at[i]], rows[b],
                                  gsem[b]).wait()
            pltpu.async_copy(rows[b], acc.at[dst_v.at[i]], ssem[b], add=True)
            nb = (b + NSLOT - 1) % NSLOT
            if b == 0:
                @pl.when(j == 0)
                def _():
                    _prepare(i + NSLOT - 1, nb, False)

                @pl.when(jnp.logical_and(j > 0, i + NSLOT - 1 < NCK))
                def _():
                    _prepare(i + NSLOT - 1, nb, True)
            else:
                @pl.when(i + NSLOT - 1 < NCK)
                def _():
                    _prepare(i + NSLOT - 1, nb, True)

    for b in range(NSLOT):
        pltpu.make_async_copy(rows[b], acc.at[dst_v.at[b]], ssem[b]).wait()

    plsc.subcore_barrier()
    pltpu.sync_copy(acc.at[pl.ds(r0, RPS)], out_hbm.at[c, pl.ds(r0, RPS)])


_RB = 2000


def _mm_body(x_ref, w_ref, o_ref):
    o_ref[...] = jnp.dot(x_ref[...], w_ref[...],
                         preferred_element_type=jnp.float32)


def _matmul(x, w):
    return pl.pallas_call(
        _mm_body,
        out_shape=jax.ShapeDtypeStruct((N, D), jnp.float32),
        grid=(N // _RB,),
        in_specs=[
            pl.BlockSpec((_RB, D), lambda i: (i, 0)),
            pl.BlockSpec((D, D), lambda i: (0, 0)),
        ],
        out_specs=pl.BlockSpec((_RB, D), lambda i: (i, 0)),
    )(x, w)


def _prep_body(deg_ref, h0_ref, dis_ref, hp_ref):
    dblk = deg_ref[...]
    total = dblk[0, :, 0:1] + dblk[1, :, 0:1] + 1.0
    dis = lax.rsqrt(total)
    dis_ref[...] = jnp.broadcast_to(dis, (_RB, D))
    hp_ref[...] = (h0_ref[...] * dis).astype(jnp.bfloat16)


def _prep(deg, h0):
    return pl.pallas_call(
        _prep_body,
        out_shape=(
            jax.ShapeDtypeStruct((N, D), jnp.float32),
            jax.ShapeDtypeStruct((N, D), jnp.bfloat16),
        ),
        grid=(N // _RB,),
        in_specs=[
            pl.BlockSpec((NC, _RB, DEG_W), lambda i: (0, i, 0)),
            pl.BlockSpec((_RB, D), lambda i: (i, 0)),
        ],
        out_specs=(
            pl.BlockSpec((_RB, D), lambda i: (i, 0)),
            pl.BlockSpec((_RB, D), lambda i: (i, 0)),
        ),
    )(deg, h0)


def _mid_body(agg_ref, hp_ref, res_ref, dis_ref, b_ref, g_ref, be_ref, w_ref,
              y_ref, hpn_ref):
    inv = 1.0 / (1.0 + BN_EPS) ** 0.5
    ablk = agg_ref[...]
    dis = dis_ref[...]
    a = ablk[0].astype(jnp.float32) + ablk[1].astype(jnp.float32)
    z = dis * (a + hp_ref[...].astype(jnp.float32)) + b_ref[...]
    z = z * (g_ref[...] * inv) + be_ref[...]
    y = jnp.maximum(z, 0.0) + res_ref[...]
    y_ref[...] = y
    h = jnp.dot(y, w_ref[...], preferred_element_type=jnp.float32)
    hpn_ref[...] = (dis * h).astype(jnp.bfloat16)


def _mid(agg, hp, res, dis, b, g, be, w):
    return pl.pallas_call(
        _mid_body,
        out_shape=(
            jax.ShapeDtypeStruct((N, D), jnp.float32),
            jax.ShapeDtypeStruct((N, D), jnp.bfloat16),
        ),
        grid=(N // _RB,),
        in_specs=[
            pl.BlockSpec((NC, _RB, D), lambda i: (0, i, 0)),
            pl.BlockSpec((_RB, D), lambda i: (i, 0)),
            pl.BlockSpec((_RB, D), lambda i: (i, 0)),
            pl.BlockSpec((_RB, D), lambda i: (i, 0)),
            pl.BlockSpec((1, D), lambda i: (0, 0)),
            pl.BlockSpec((1, D), lambda i: (0, 0)),
            pl.BlockSpec((1, D), lambda i: (0, 0)),
            pl.BlockSpec((D, D), lambda i: (0, 0)),
        ],
        out_specs=(
            pl.BlockSpec((_RB, D), lambda i: (i, 0)),
            pl.BlockSpec((_RB, D), lambda i: (i, 0)),
        ),
    )(agg, hp, res, dis, b, g, be, w)


def _final_body(agg_ref, hp_ref, dis_ref, b_ref, o_ref):
    ablk = agg_ref[...]
    a = ablk[0].astype(jnp.float32) + ablk[1].astype(jnp.float32)
    o_ref[...] = (dis_ref[...] * (a + hp_ref[...].astype(jnp.float32))
                  + b_ref[...])


def _final(agg, hp, dis, b):
    return pl.pallas_call(
        _final_body,
        out_shape=jax.ShapeDtypeStruct((N, D), jnp.float32),
        grid=(N // _RB,),
        in_specs=[
            pl.BlockSpec((NC, _RB, D), lambda i: (0, i, 0)),
            pl.BlockSpec((_RB, D), lambda i: (i, 0)),
            pl.BlockSpec((_RB, D), lambda i: (i, 0)),
            pl.BlockSpec((1, D), lambda i: (0, 0)),
        ],
        out_specs=pl.BlockSpec((_RB, D), lambda i: (i, 0)),
    )(agg, hp, dis, b)


def kernel(x, edge_index, W0, b0, W1, b1, W2, b2, g0, be0, g1, be1):
    edge_index = edge_index.astype(jnp.int32)
    src = edge_index[0]
    dst = edge_index[1]
    npad = EPAD - E
    pad_iota = jnp.arange(npad, dtype=jnp.int32) % CHUNK
    srcR = jnp.concatenate([src, pad_iota]).reshape(NW, NCK, CHUNK)
    dstR = jnp.concatenate([dst, N + pad_iota]).reshape(NW, NCK, CHUNK)

    zeros_deg = jnp.zeros((NPAD, DEG_W), jnp.float32)
    ones_rows = jnp.ones((CHUNK, DEG_W), jnp.float32)
    b0r = b0.reshape(1, D)
    b1r = b1.reshape(1, D)
    b2r = b2.reshape(1, D)
    g0r = g0.reshape(1, D)
    g1r = g1.reshape(1, D)
    be0r = be0.reshape(1, D)
    be1r = be1.reshape(1, D)

    deg = _deg_kernel(dstR, ones_rows, zeros_deg)
    h0 = _matmul(x, W0)
    dis, hp0 = _prep(deg, h0)

    agg0 = _agg_kernel(srcR, dstR, hp0)
    y1, hp1 = _mid(agg0, hp0, x, dis, b0r, g0r, be0r, W1)

    agg1 = _agg_kernel(srcR, dstR, hp1)
    y2, hp2 = _mid(agg1, hp1, y1, dis, b1r, g1r, be1r, W2)

    agg2 = _agg_kernel(srcR, dstR, hp2)
    return _final(agg2, hp2, dis, b2r)


# speedup vs baseline: 1.0097x; 1.0097x over previous
import functools

import jax
import jax.numpy as jnp
from jax import lax
from jax.experimental import pallas as pl
from jax.experimental.pallas import tpu as pltpu
from jax.experimental.pallas import tpu_sc as plsc

N = 10000
E = 320000
D = 128
BN_EPS = 1e-5

NC = 2
NS = 16
NW = NC * NS
CHUNK = 128
NCK = 80
EPAD = NW * NCK * CHUNK
NR = N + CHUNK
RPS = N // NS
DEG_W = 16
NPAD = 10240
DROWS = NPAD // NS
NSLOT = 8

_mesh = plsc.VectorSubcoreMesh(core_axis_name="c", subcore_axis_name="s")
_sc_params = pltpu.CompilerParams(use_tc_tiling_on_sc=False)


@functools.partial(
    pl.kernel,
    out_type=jax.ShapeDtypeStruct((NC, NPAD, DEG_W), jnp.float32),
    mesh=_mesh,
    scratch_types=[
        pltpu.VMEM((NCK, CHUNK), jnp.int32),
        pltpu.VMEM((CHUNK, DEG_W), jnp.float32),
        pltpu.VMEM_SHARED((NPAD, DEG_W), jnp.float32),
        pltpu.SemaphoreType.DMA,
    ],
    compiler_params=_sc_params,
)
def _deg_kernel(dstR_hbm, ones_hbm, zeros_hbm, out_hbm, dst_v, ones_v, acc,
                sem):
    c = lax.axis_index("c")
    s = lax.axis_index("s")
    wid = c * NS + s

    pltpu.sync_copy(ones_hbm, ones_v)
    pltpu.sync_copy(dstR_hbm.at[wid], dst_v)
    r0 = s * DROWS
    pltpu.sync_copy(zeros_hbm.at[pl.ds(r0, DROWS)], acc.at[pl.ds(r0, DROWS)])
    plsc.subcore_barrier()

    @pl.loop(0, NCK // 8)
    def _wave(w):
        i0 = w * 8
        for k in range(8):
            pltpu.async_copy(ones_v, acc.at[dst_v.at[i0 + k]], sem, add=True)
        for k in range(8):
            pltpu.make_async_copy(ones_v, acc.at[dst_v.at[i0 + k]], sem).wait()

    plsc.subcore_barrier()
    pltpu.sync_copy(acc.at[pl.ds(r0, DROWS)], out_hbm.at[c, pl.ds(r0, DROWS)])


@functools.partial(
    pl.kernel,
    out_type=jax.ShapeDtypeStruct((NC, N, D), jnp.bfloat16),
    mesh=_mesh,
    scratch_types=[
        pltpu.VMEM((NCK, CHUNK), jnp.int32),
        pltpu.VMEM((NCK, CHUNK), jnp.int32),
        [pltpu.VMEM((CHUNK, D), jnp.bfloat16) for _ in range(NSLOT)],
        pltpu.VMEM_SHARED((NR, D), jnp.bfloat16),
        [pltpu.SemaphoreType.DMA for _ in range(NSLOT)],
        [pltpu.SemaphoreType.DMA for _ in range(NSLOT)],
    ],
    compiler_params=_sc_params,
)
def _agg_kernel(srcR_hbm, dstR_hbm, hp_hbm, out_hbm,
                src_v, dst_v, rows, acc, gsem, ssem):
    c = lax.axis_index("c")
    s = lax.axis_index("s")
    wid = c * NS + s

    zero32 = jnp.zeros((32,), jnp.bfloat16)

    @pl.loop(0, CHUNK)
    def _zfill(r):
        for k in range(D // 32):
            rows[0][r, pl.ds(k * 32, 32)] = zero32

    r0 = s * RPS
    for k in range(4):
        pltpu.sync_copy(rows[0], acc.at[pl.ds(r0 + k * CHUNK, CHUNK)])
    pltpu.sync_copy(rows[0].at[pl.ds(0, RPS - 4 * CHUNK)],
                    acc.at[pl.ds(r0 + 4 * CHUNK, RPS - 4 * CHUNK)])

    pltpu.sync_copy(srcR_hbm.at[wid], src_v)
    pltpu.sync_copy(dstR_hbm.at[wid], dst_v)

    for b in range(NSLOT - 1):
        pltpu.async_copy(hp_hbm.at[src_v.at[b]], rows[b], gsem[b])
    plsc.subcore_barrier()

    def _prepare(i_next, b_next, wait_prev):
        if wait_prev:
            pltpu.make_async_copy(rows[b_next],
                                  acc.at[dst_v.at[i_next]],
                                  ssem[b_next]).wait()
        pltpu.async_copy(hp_hbm.at[src_v.at[i_next]], rows[b_next],
                         gsem[b_next])

    @pl.loop(0, NCK // NSLOT)
    def _group(j):
        base = j * NSLOT
        for b in range(NSLOT):
            i = base + b
            pltpu.make_async_copy(hp_hbm.at[src_v.at[i]], rows[b],
                                  gsem[b]).wait()
            pltpu.async_copy(rows[b], acc.at[dst_v.at[i]], ssem[b], add=True)
            nb = (b + NSLOT - 1) % NSLOT
            if b == 0:
                @pl.when(j == 0)
                def _():
                    _prepare(i + NSLOT - 1, nb, False)

                @pl.when(jnp.logical_and(j > 0, i + NSLOT - 1 < NCK))
                def _():
                    _prepare(i + NSLOT - 1, nb, True)
            else:
                @pl.when(i + NSLOT - 1 < NCK)
                def _():
                    _prepare(i + NSLOT - 1, nb, True)

    for b in range(NSLOT):
        pltpu.make_async_copy(rows[b], acc.at[dst_v.at[b]], ssem[b]).wait()

    plsc.subcore_barrier()
    pltpu.sync_copy(acc.at[pl.ds(r0, RPS)], out_hbm.at[c, pl.ds(r0, RPS)])


_RB = 2000


def _mm_body(x_ref, w_ref, o_ref):
    o_ref[...] = jnp.dot(x_ref[...], w_ref[...],
                         preferred_element_type=jnp.float32)


def _matmul(x, w):
    return pl.pallas_call(
        _mm_body,
        out_shape=jax.ShapeDtypeStruct((N, D), jnp.float32),
        grid=(N // _RB,),
        in_specs=[
            pl.BlockSpec((_RB, D), lambda i: (i, 0)),
            pl.BlockSpec((D, D), lambda i: (0, 0)),
        ],
        out_specs=pl.BlockSpec((_RB, D), lambda i: (i, 0)),
    )(x, w)


def _prep_body(deg_ref, h0_ref, dis_ref, hp_ref):
    dblk = deg_ref[...]
    total = dblk[0, :, 0:1] + dblk[1, :, 0:1] + 1.0
    dis = lax.rsqrt(total)
    dis_ref[...] = jnp.broadcast_to(dis, (_RB, D))
    hp_ref[...] = (h0_ref[...] * dis).astype(jnp.bfloat16)


def _prep(deg, h0):
    return pl.pallas_call(
        _prep_body,
        out_shape=(
            jax.ShapeDtypeStruct((N, D), jnp.float32),
            jax.ShapeDtypeStruct((N, D), jnp.bfloat16),
        ),
        grid=(N // _RB,),
        in_specs=[
            pl.BlockSpec((NC, _RB, DEG_W), lambda i: (0, i, 0)),
            pl.BlockSpec((_RB, D), lambda i: (i, 0)),
        ],
        out_specs=(
            pl.BlockSpec((_RB, D), lambda i: (i, 0)),
            pl.BlockSpec((_RB, D), lambda i: (i, 0)),
        ),
    )(deg, h0)


def _mid_body(agg_ref, hp_ref, res_ref, dis_ref, b_ref, g_ref, be_ref, w_ref,
              y_ref, hpn_ref):
    inv = 1.0 / (1.0 + BN_EPS) ** 0.5
    ablk = agg_ref[...]
    dis = dis_ref[...]
    a = ablk[0].astype(jnp.float32) + ablk[1].astype(jnp.float32)
    z = dis * (a + hp_ref[...].astype(jnp.float32)) + b_ref[...]
    z = z * (g_ref[...] * inv) + be_ref[...]
    y = jnp.maximum(z, 0.0) + res_ref[...]
    y_ref[...] = y
    h = jnp.dot(y, w_ref[...], preferred_element_type=jnp.float32)
    hpn_ref[...] = (dis * h).astype(jnp.bfloat16)


def _mid(agg, hp, res, dis, b, g, be, w):
    return pl.pallas_call(
        _mid_body,
        out_shape=(
            jax.ShapeDtypeStruct((N, D), jnp.float32),
            jax.ShapeDtypeStruct((N, D), jnp.bfloat16),
        ),
        grid=(N // _RB,),
        in_specs=[
            pl.BlockSpec((NC, _RB, D), lambda i: (0, i, 0)),
            pl.BlockSpec((_RB, D), lambda i: (i, 0)),
            pl.BlockSpec((_RB, D), lambda i: (i, 0)),
            pl.BlockSpec((_RB, D), lambda i: (i, 0)),
            pl.BlockSpec((1, D), lambda i: (0, 0)),
            pl.BlockSpec((1, D), lambda i: (0, 0)),
            pl.BlockSpec((1, D), lambda i: (0, 0)),
            pl.BlockSpec((D, D), lambda i: (0, 0)),
        ],
        out_specs=(
            pl.BlockSpec((_RB, D), lambda i: (i, 0)),
            pl.BlockSpec((_RB, D), lambda i: (i, 0)),
        ),
    )(agg, hp, res, dis, b, g, be, w)


def _final_body(agg_ref, hp_ref, dis_ref, b_ref, o_ref):
    ablk = agg_ref[...]
    a = ablk[0].astype(jnp.float32) + ablk[1].astype(jnp.float32)
    o_ref[...] = (dis_ref[...] * (a + hp_ref[...].astype(jnp.float32))
                  + b_ref[...])


def _final(agg, hp, dis, b):
    return pl.pallas_call(
        _final_body,
        out_shape=jax.ShapeDtypeStruct((N, D), jnp.float32),
        grid=(N // _RB,),
        in_specs=[
            pl.BlockSpec((NC, _RB, D), lambda i: (0, i, 0)),
            pl.BlockSpec((_RB, D), lambda i: (i, 0)),
            pl.BlockSpec((_RB, D), lambda i: (i, 0)),
            pl.BlockSpec((1, D), lambda i: (0, 0)),
        ],
        out_specs=pl.BlockSpec((_RB, D), lambda i: (i, 0)),
    )(agg, hp, dis, b)


def kernel(x, edge_index, W0, b0, W1, b1, W2, b2, g0, be0, g1, be1):
    edge_index = edge_index.astype(jnp.int32)
    src = edge_index[0]
    dst = edge_index[1]
    npad = EPAD - E
    pad_iota = jnp.arange(npad, dtype=jnp.int32) % CHUNK
    srcR = jnp.concatenate([src, pad_iota]).reshape(NW, NCK, CHUNK)
    dstR = jnp.concatenate([dst, N + pad_iota]).reshape(NW, NCK, CHUNK)

    zeros_deg = jnp.zeros((NPAD, DEG_W), jnp.float32)
    ones_rows = jnp.ones((CHUNK, DEG_W), jnp.float32)
    b0r = b0.reshape(1, D)
    b1r = b1.reshape(1, D)
    b2r = b2.reshape(1, D)
    g0r = g0.reshape(1, D)
    g1r = g1.reshape(1, D)
    be0r = be0.reshape(1, D)
    be1r = be1.reshape(1, D)

    deg = _deg_kernel(dstR, ones_rows, zeros_deg)
    h0 = _matmul(x, W0)
    dis, hp0 = _prep(deg, h0)

    agg0 = _agg_kernel(srcR, dstR, hp0)
    y1, hp1 = _mid(agg0, hp0, x, dis, b0r, g0r, be0r, W1)

    agg1 = _agg_kernel(srcR, dstR, hp1)
    y2, hp2 = _mid(agg1, hp1, y1, dis, b1r, g1r, be1r, W2)

    agg2 = _agg_kernel(srcR, dstR, hp2)
    return _final(agg2, hp2, dis, b2r)
